# baseline (speedup 1.0000x reference)
import functools

import jax
import jax.numpy as jnp
from jax import lax
from jax.experimental import pallas as pl
from jax.experimental.pallas import tpu as pltpu
from jax.experimental.pallas import tpu_sc as plsc

NUM_NODES = 100000
H_DIM = 64
BATCH = 16384

_NC = 2
_NS = 16
_NW = _NC * _NS
_B_PER_W = BATCH // _NW
_C = 4
_CH = _B_PER_W // _C


def _make_gather():
    mesh = plsc.VectorSubcoreMesh(core_axis_name="c", subcore_axis_name="s")

    @functools.partial(
        pl.kernel,
        mesh=mesh,
        compiler_params=pltpu.CompilerParams(use_tc_tiling_on_sc=False),
        out_type=jax.ShapeDtypeStruct((BATCH, H_DIM), jnp.float32),
        scratch_types=[
            pltpu.VMEM((_B_PER_W,), jnp.int32),
            pltpu.VMEM((_C, _CH, H_DIM), jnp.float32),
            pltpu.SemaphoreType.DMA,
            pltpu.SemaphoreType.DMA,
        ],
    )
    def gather_kernel(idx_hbm, table_hbm, out_hbm, idx_v, rows_v, gsem, wsem):
        wid = lax.axis_index("s") * _NC + lax.axis_index("c")
        base = wid * _B_PER_W
        pltpu.sync_copy(idx_hbm.at[pl.ds(base, _B_PER_W)], idx_v)
        gathers = [
            pltpu.async_copy(
                table_hbm.at[idx_v.at[pl.ds(c * _CH, _CH)]],
                rows_v.at[c],
                gsem,
            )
            for c in range(_C)
        ]
        writebacks = []
        for c in range(_C):
            gathers[c].wait()
            writebacks.append(
                pltpu.async_copy(
                    rows_v.at[c],
                    out_hbm.at[pl.ds(base + c * _CH, _CH)],
                    wsem,
                )
            )
        for wb in writebacks:
            wb.wait()

    return gather_kernel


_gather = _make_gather()


def kernel(g, h, table):
    idx = h.reshape(BATCH)
    return _gather(idx, table)


# speedup vs baseline: 1.0792x; 1.0792x over previous
import functools

import jax
import jax.numpy as jnp
from jax import lax
from jax.experimental import pallas as pl
from jax.experimental.pallas import tpu as pltpu
from jax.experimental.pallas import tpu_sc as plsc

NUM_NODES = 100000
H_DIM = 64
BATCH = 16384
OUT_W = 128

_NC = 2
_NS = 16
_NW = _NC * _NS
_B_PER_W = BATCH // _NW
_C = 4
_CH = _B_PER_W // _C


def _make_gather():
    mesh = plsc.VectorSubcoreMesh(core_axis_name="c", subcore_axis_name="s")

    @functools.partial(
        pl.kernel,
        mesh=mesh,
        compiler_params=pltpu.CompilerParams(use_tc_tiling_on_sc=False),
        out_type=jax.ShapeDtypeStruct((BATCH, OUT_W), jnp.float32),
        scratch_types=[
            pltpu.VMEM((_B_PER_W,), jnp.int32),
            pltpu.VMEM((_C, _CH, H_DIM), jnp.float32),
            pltpu.SemaphoreType.DMA,
            pltpu.SemaphoreType.DMA,
        ],
    )
    def gather_kernel(idx_hbm, table_hbm, out_hbm, idx_v, rows_v, gsem, wsem):
        wid = lax.axis_index("s") * _NC + lax.axis_index("c")
        base = wid * _B_PER_W
        pltpu.sync_copy(idx_hbm.at[pl.ds(base, _B_PER_W)], idx_v)
        gathers = [
            pltpu.async_copy(
                table_hbm.at[idx_v.at[pl.ds(c * _CH, _CH)]],
                rows_v.at[c],
                gsem,
            )
            for c in range(_C)
        ]
        writebacks = []
        for c in range(_C):
            gathers[c].wait()
            writebacks.append(
                pltpu.async_copy(
                    rows_v.at[c],
                    out_hbm.at[pl.ds(base + c * _CH, _CH), pl.ds(0, H_DIM)],
                    wsem,
                )
            )
        for wb in writebacks:
            wb.wait()

    return gather_kernel


_gather = _make_gather()


def kernel(g, h, table):
    idx = lax.optimization_barrier(h.reshape(BATCH))
    out_padded = _gather(idx, table)
    return out_padded[:, :H_DIM]
